# baseline (speedup 1.0000x reference)
import functools

import jax
import jax.numpy as jnp
from jax.experimental import pallas as pl
from jax.experimental.pallas import tpu as pltpu

_NP_RATIO = 3
_VAR0, _VAR1 = 0.1, 0.2
_FPB = 8
_NCHUNK = 4


def _smooth_l1(x):
    ax = jnp.abs(x)
    return jnp.where(ax < 1.0, 0.5 * x * x, ax - 0.5)


def _match_kernel(tgt_ref, anc_ref, loc_ref, conf_ref,
                  out_l_ref, out_c_ref, ce_ref, np_ref, *, n_anchors, n_objs):
    step = pl.program_id(0)
    A = n_anchors
    O = n_objs
    P = _FPB

    @pl.when(step == 0)
    def _init():
        out_l_ref[:, :] = jnp.zeros((1, 1), jnp.float32)
        out_c_ref[:, :] = jnp.zeros((1, 1), jnp.float32)

    anc = anc_ref[:, :]
    cx, cy = anc[0:1, :], anc[1:2, :]
    w, h = anc[2:3, :], anc[3:4, :]
    ax1 = (cx - w * 0.5)[None]
    ay1 = (cy - h * 0.5)[None]
    ax2 = (cx + w * 0.5)[None]
    ay2 = (cy + h * 0.5)[None]
    area_a = (w * h)[None]

    tgt = tgt_ref[:, :, :]
    tx1, ty1 = tgt[:, :, 0:1], tgt[:, :, 1:2]
    tx2, ty2 = tgt[:, :, 2:3], tgt[:, :, 3:4]
    area_t = (tx2 - tx1) * (ty2 - ty1)

    iw = jnp.minimum(tx2, ax2) - jnp.maximum(tx1, ax1)
    ih = jnp.minimum(ty2, ay2) - jnp.maximum(ty1, ay1)
    inter = jnp.maximum(iw, 0.0) * jnp.maximum(ih, 0.0)
    ov = inter / (area_t + area_a - inter)

    o_iota = jax.lax.broadcasted_iota(jnp.int32, (1, O, 1), 1)
    a_iota = jax.lax.broadcasted_iota(jnp.int32, (1, 1, A), 2)

    BITS2 = 0x40000000
    BITS_HALF = 0x3F000000
    ovb = jax.lax.bitcast_convert_type(ov, jnp.int32)
    key_n = (ovb & ~0xF) | (15 - o_iota)

    m_t = jnp.max(ov, axis=2, keepdims=True)
    bpi = jnp.min(jnp.where(ov == m_t, a_iota, A), axis=2, keepdims=True)
    forced = bpi == a_iota
    key = jnp.where(forced, BITS2 + o_iota, key_n)
    kmax = jnp.max(key, axis=1)

    is_f = kmax >= BITS2
    low = kmax & 0xF
    bti = jnp.where(is_f, low, 15 - low)
    pos = (kmax & ~0xF) >= BITS_HALF

    b0 = (bti & 1) != 0
    b1 = (bti & 2) != 0
    b2 = (bti & 4) != 0
    b3 = (bti & 8) != 0

    def _tree(tc):
        v = [jnp.where(b0, tc[:, 2 * j + 1, :], tc[:, 2 * j, :])
             for j in range(8)]
        v = [jnp.where(b1, v[2 * j + 1], v[2 * j]) for j in range(4)]
        v = [jnp.where(b2, v[2 * j + 1], v[2 * j]) for j in range(2)]
        return jnp.where(b3, v[1], v[0])

    mx1, my1 = _tree(tx1), _tree(ty1)
    mx2, my2 = _tree(tx2), _tree(ty2)

    cx1, cy1 = cx, cy
    g0 = ((mx1 + mx2) * 0.5 - cx1) / (_VAR0 * w)
    g1 = ((my1 + my2) * 0.5 - cy1) / (_VAR0 * h)
    g2 = jnp.log((mx2 - mx1) / w) / _VAR1
    g3 = jnp.log((my2 - my1) / h) / _VAR1

    loc = loc_ref[:, :, :]
    sl = (_smooth_l1(loc[:, 0, :] - g0) + _smooth_l1(loc[:, 1, :] - g1) +
          _smooth_l1(loc[:, 2, :] - g2) + _smooth_l1(loc[:, 3, :] - g3))
    lsum = jnp.sum(jnp.where(pos, sl, 0.0), axis=1, keepdims=True)
    out_l_ref[:, :] += jnp.sum(lsum, axis=0, keepdims=True)

    c0 = conf_ref[:, 0, :]
    c1 = conf_ref[:, 1, :]
    lse = jnp.maximum(c0, c1) + jnp.log(1.0 + jnp.exp(-jnp.abs(c0 - c1)))
    ce = lse - jnp.where(pos, c1, c0)
    csum = jnp.sum(jnp.where(pos, ce, 0.0), axis=1, keepdims=True)
    out_c_ref[:, :] += jnp.sum(csum, axis=0, keepdims=True)

    ce_ref[:, :] = jnp.where(pos, 0.0, ce)
    n_pos = jnp.sum(pos.astype(jnp.int32), axis=1, keepdims=True)
    np_ref[:, :] = jnp.broadcast_to(n_pos, (P, 128))


def _mine_kernel(lp_ref, cp_ref, np0_ref, np1_ref, np2_ref, np3_ref,
                 ce0_ref, ce1_ref, ce2_ref, ce3_ref,
                 out_l_ref, out_c_ref, *, n_frames, n_anchors):
    A = n_anchors
    ce_refs = (ce0_ref, ce1_ref, ce2_ref, ce3_ref)
    np_refs = (np0_ref, np1_ref, np2_ref, np3_ref)
    CF = n_frames // _NCHUNK

    npos = jnp.concatenate([r[:, 0:1] for r in np_refs], axis=0)
    K = jnp.minimum(npos * _NP_RATIO, A - 1)

    def body(i, t):
        bit = jax.lax.shift_left(jnp.int32(1), jnp.int32(30) - i)
        cand = t + bit
        cnts = []
        for j, r in enumerate(ce_refs):
            bits = jax.lax.bitcast_convert_type(r[:, :], jnp.int32)
            cj = cand[j * CF:(j + 1) * CF, :]
            cnts.append(jnp.sum((bits >= cj).astype(jnp.int32), axis=1,
                                keepdims=True))
        cnt = jnp.concatenate(cnts, axis=0)
        return jnp.where(cnt >= K, cand, t)

    t0 = jnp.zeros((n_frames, 1), jnp.int32)
    t = jax.lax.fori_loop(0, 31, body, t0)
    tf = jax.lax.bitcast_convert_type(t, jnp.float32)

    tops = []
    for j, r in enumerate(ce_refs):
        V = r[:, :]
        tj = tf[j * CF:(j + 1) * CF, :]
        gt = V > tj
        cnt_gt = jnp.sum(jnp.where(gt, 1.0, 0.0), axis=1, keepdims=True)
        sum_gt = jnp.sum(jnp.where(gt, V, 0.0), axis=1, keepdims=True)
        Kj = K[j * CF:(j + 1) * CF, :].astype(jnp.float32)
        top = sum_gt + (Kj - cnt_gt) * tj
        tops.append(jnp.where(Kj > 0, top, 0.0))
    top_all = jnp.concatenate(tops, axis=0)

    out_l_ref[:, :] = jnp.sum(lp_ref[:, :], axis=0, keepdims=True)
    out_c_ref[:, :] = (jnp.sum(cp_ref[:, :], axis=0, keepdims=True) +
                       jnp.sum(top_all, axis=0, keepdims=True))


def kernel(loc_data, conf_data, anchors, targets):
    B = targets.shape[0]
    F = targets.shape[1]
    O = targets.shape[2]
    A = anchors.shape[0]
    BF = B * F
    P = _FPB
    NC = _NCHUNK
    CF = BF // NC

    loc_r = loc_data.reshape(BF, A, 4)
    conf_r = conf_data.reshape(BF, A, 2)
    tgt_r = targets.reshape(BF, O, 5)
    anc_t = anchors.T

    match = pl.pallas_call(
        functools.partial(_match_kernel, n_anchors=A, n_objs=O),
        grid=(CF // P,),
        in_specs=[
            pl.BlockSpec((P, O, 5), lambda i: (i, 0, 0)),
            pl.BlockSpec((4, A), lambda i: (0, 0)),
            pl.BlockSpec((P, 4, A), lambda i: (i, 0, 0)),
            pl.BlockSpec((P, 2, A), lambda i: (i, 0, 0)),
        ],
        out_specs=[
            pl.BlockSpec((1, 1), lambda i: (0, 0)),
            pl.BlockSpec((1, 1), lambda i: (0, 0)),
            pl.BlockSpec((P, A), lambda i: (i, 0)),
            pl.BlockSpec((P, 128), lambda i: (i, 0)),
        ],
        out_shape=[
            jax.ShapeDtypeStruct((1, 1), jnp.float32),
            jax.ShapeDtypeStruct((1, 1), jnp.float32),
            jax.ShapeDtypeStruct((CF, A), jnp.float32),
            jax.ShapeDtypeStruct((CF, 128), jnp.int32),
        ],
    )

    lps, cps, ces, nps = [], [], [], []
    for c in range(NC):
        sl = slice(c * CF, (c + 1) * CF)
        loc_c = loc_r[sl].transpose(0, 2, 1)
        conf_c = conf_r[sl].transpose(0, 2, 1)
        l_c, c_c, ce_c, np_c = match(tgt_r[sl], anc_t, loc_c, conf_c)
        lps.append(l_c)
        cps.append(c_c)
        ces.append(ce_c)
        nps.append(np_c)

    lp = jnp.concatenate(lps, axis=0)
    cp = jnp.concatenate(cps, axis=0)

    out_l, out_c = pl.pallas_call(
        functools.partial(_mine_kernel, n_frames=BF, n_anchors=A),
        grid=(1,),
        in_specs=[
            pl.BlockSpec((NC, 1), lambda i: (0, 0)),
            pl.BlockSpec((NC, 1), lambda i: (0, 0)),
        ] + [pl.BlockSpec((CF, 128), lambda i: (0, 0))] * NC
          + [pl.BlockSpec((CF, A), lambda i: (0, 0))] * NC,
        out_specs=[
            pl.BlockSpec((1, 1), lambda i: (0, 0)),
            pl.BlockSpec((1, 1), lambda i: (0, 0)),
        ],
        out_shape=[
            jax.ShapeDtypeStruct((1, 1), jnp.float32),
            jax.ShapeDtypeStruct((1, 1), jnp.float32),
        ],
    )(lp, cp, *nps, *ces)
    return (out_l[0, 0], out_c[0, 0])


# speedup vs baseline: 1.2106x; 1.2106x over previous
import functools

import jax
import jax.numpy as jnp
from jax.experimental import pallas as pl
from jax.experimental.pallas import tpu as pltpu

_NP_RATIO = 3
_THRESHOLD = 0.5
_VAR0, _VAR1 = 0.1, 0.2


def _smooth_l1(x):
    ax = jnp.abs(x)
    return jnp.where(ax < 1.0, 0.5 * x * x, ax - 0.5)


def _loss_kernel(tgt_ref, anc_ref, loc_ref, conf_ref, out_l_ref, out_c_ref,
                 ce_ref, np_ref, *, n_frames, n_anchors, n_objs):
    bf = pl.program_id(0)
    A = n_anchors
    O = n_objs

    @pl.when(bf == 0)
    def _init():
        out_l_ref[:, :] = jnp.zeros((1, 1), jnp.float32)
        out_c_ref[:, :] = jnp.zeros((1, 1), jnp.float32)

    anc = anc_ref[:, :]
    cx, cy, w, h = anc[0:1, :], anc[1:2, :], anc[2:3, :], anc[3:4, :]
    ax1, ay1 = cx - w * 0.5, cy - h * 0.5
    ax2, ay2 = cx + w * 0.5, cy + h * 0.5
    area_a = w * h

    tgt = tgt_ref[0]
    tx1, ty1 = tgt[:, 0:1], tgt[:, 1:2]
    tx2, ty2 = tgt[:, 2:3], tgt[:, 3:4]
    area_t = (tx2 - tx1) * (ty2 - ty1)

    iw = jnp.minimum(tx2, ax2) - jnp.maximum(tx1, ax1)
    ih = jnp.minimum(ty2, ay2) - jnp.maximum(ty1, ay1)
    inter = jnp.maximum(iw, 0.0) * jnp.maximum(ih, 0.0)
    ov = inter / (area_t + area_a - inter)

    o_iota = jax.lax.broadcasted_iota(jnp.int32, (O, 1), 0)
    a_iota = jax.lax.broadcasted_iota(jnp.int32, (1, A), 1)

    bto = jnp.max(ov, axis=0, keepdims=True)
    bti = jnp.min(jnp.where(ov == bto, o_iota, O), axis=0, keepdims=True)

    m_t = jnp.max(ov, axis=1, keepdims=True)
    bpi = jnp.min(jnp.where(ov == m_t, a_iota, A), axis=1, keepdims=True)
    forced = bpi == a_iota
    f_idx = jnp.max(jnp.where(forced, o_iota, -1), axis=0, keepdims=True)
    is_f = f_idx >= 0
    bto = jnp.where(is_f, 2.0, bto)
    bti = jnp.where(is_f, f_idx, bti)

    sel_t = bti == o_iota
    mx1 = jnp.sum(jnp.where(sel_t, tx1, 0.0), axis=0, keepdims=True)
    my1 = jnp.sum(jnp.where(sel_t, ty1, 0.0), axis=0, keepdims=True)
    mx2 = jnp.sum(jnp.where(sel_t, tx2, 0.0), axis=0, keepdims=True)
    my2 = jnp.sum(jnp.where(sel_t, ty2, 0.0), axis=0, keepdims=True)

    pos = jnp.logical_not(bto < _THRESHOLD)

    g0 = ((mx1 + mx2) * 0.5 - cx) / (_VAR0 * w)
    g1 = ((my1 + my2) * 0.5 - cy) / (_VAR0 * h)
    g2 = jnp.log((mx2 - mx1) / w) / _VAR1
    g3 = jnp.log((my2 - my1) / h) / _VAR1

    loc = loc_ref[0]
    sl = (_smooth_l1(loc[0:1, :] - g0) + _smooth_l1(loc[1:2, :] - g1) +
          _smooth_l1(loc[2:3, :] - g2) + _smooth_l1(loc[3:4, :] - g3))
    out_l_ref[:, :] += jnp.sum(jnp.where(pos, sl, 0.0), axis=1, keepdims=True)

    conf = conf_ref[0]
    c0, c1 = conf[0:1, :], conf[1:2, :]
    lse = jnp.maximum(c0, c1) + jnp.log(1.0 + jnp.exp(-jnp.abs(c0 - c1)))
    ce = lse - jnp.where(pos, c1, c0)
    out_c_ref[:, :] += jnp.sum(jnp.where(pos, ce, 0.0), axis=1, keepdims=True)

    ce_ref[pl.ds(bf, 1), :] = jnp.where(pos, 0.0, ce)
    n_pos = jnp.sum(pos.astype(jnp.int32), axis=1, keepdims=True)
    np_ref[pl.ds(bf, 1), :] = jnp.broadcast_to(n_pos, (1, 128))

    @pl.when(bf == n_frames - 1)
    def _tail():
        npos = np_ref[:, 0:1]
        K = jnp.minimum(npos * _NP_RATIO, A - 1)

        def body(i, t):
            bit = jax.lax.shift_left(jnp.int32(1), jnp.int32(30) - i)
            cand = t + bit
            bits = jax.lax.bitcast_convert_type(ce_ref[:, :], jnp.int32)
            cnt = jnp.sum((bits >= cand).astype(jnp.int32), axis=1,
                          keepdims=True)
            return jnp.where(cnt >= K, cand, t)

        t0 = jnp.zeros((n_frames, 1), jnp.int32)
        t = jax.lax.fori_loop(0, 31, body, t0)
        tf = jax.lax.bitcast_convert_type(t, jnp.float32)
        V = ce_ref[:, :]
        gt = V > tf
        cnt_gt = jnp.sum(jnp.where(gt, 1.0, 0.0), axis=1, keepdims=True)
        sum_gt = jnp.sum(jnp.where(gt, V, 0.0), axis=1, keepdims=True)
        top = sum_gt + (K.astype(jnp.float32) - cnt_gt) * tf
        top = jnp.where(K > 0, top, 0.0)
        out_c_ref[:, :] += jnp.sum(top, axis=0, keepdims=True)


def kernel(loc_data, conf_data, anchors, targets):
    B = targets.shape[0]
    F = targets.shape[1]
    O = targets.shape[2]
    A = anchors.shape[0]
    BF = B * F

    loc_p = loc_data.reshape(BF, A, 4).transpose(0, 2, 1)
    conf_p = conf_data.reshape(BF, A, 2).transpose(0, 2, 1)
    tgt = targets.reshape(BF, O, 5)
    anc_t = anchors.T

    out_l, out_c = pl.pallas_call(
        functools.partial(_loss_kernel, n_frames=BF, n_anchors=A, n_objs=O),
        grid=(BF,),
        in_specs=[
            pl.BlockSpec((1, O, 5), lambda i: (i, 0, 0)),
            pl.BlockSpec((4, A), lambda i: (0, 0)),
            pl.BlockSpec((1, 4, A), lambda i: (i, 0, 0)),
            pl.BlockSpec((1, 2, A), lambda i: (i, 0, 0)),
        ],
        out_specs=[
            pl.BlockSpec((1, 1), lambda i: (0, 0)),
            pl.BlockSpec((1, 1), lambda i: (0, 0)),
        ],
        out_shape=[
            jax.ShapeDtypeStruct((1, 1), jnp.float32),
            jax.ShapeDtypeStruct((1, 1), jnp.float32),
        ],
        scratch_shapes=[
            pltpu.VMEM((BF, A), jnp.float32),
            pltpu.VMEM((BF, 128), jnp.int32),
        ],
    )(tgt, anc_t, loc_p, conf_p)
    return (out_l[0, 0], out_c[0, 0])
